# baseline (speedup 1.0000x reference)
import jax
import jax.numpy as jnp
from jax.experimental import pallas as pl

NUM_TOKENS = 32768
HIDDEN = 2048
NUM_EXPERTS = 64
TOP_K = 8
BLOCK = 512
N_BLOCKS = NUM_TOKENS // BLOCK


def _fused_kernel(h_ref, logits_ref, h_out_ref, w_ref, id_ref):
    h_out_ref[...] = h_ref[...]

    i = pl.program_id(0)
    x = logits_ref[pl.ds(i * BLOCK, BLOCK), :]
    xt = x.T
    b = xt.shape[1]
    mx = jnp.max(xt, axis=0, keepdims=True)
    e = jnp.exp(xt - mx)
    probs = e / jnp.sum(e, axis=0, keepdims=True)

    row8 = jax.lax.broadcasted_iota(jnp.int32, (TOP_K, b), 0)
    row64 = jax.lax.broadcasted_iota(jnp.int32, (NUM_EXPERTS, b), 0)
    vals = jnp.zeros((TOP_K, b), dtype=jnp.float32)
    ids = jnp.zeros((TOP_K, b), dtype=jnp.int32)
    cur = probs
    for j in range(TOP_K):
        m = jnp.max(cur, axis=0, keepdims=True)
        a = jnp.argmax(cur, axis=0).astype(jnp.int32)
        a2 = a[None, :]
        vals = jnp.where(row8 == j, m, vals)
        ids = jnp.where(row8 == j, a2, ids)
        cur = jnp.where(row64 == a2, -1.0, cur)
    w = vals / jnp.sum(vals, axis=0, keepdims=True)
    w_ref[:, pl.ds(i * BLOCK, BLOCK)] = w
    id_ref[:, pl.ds(i * BLOCK, BLOCK)] = ids


def kernel(hidden_states, router_logits):
    grid = (N_BLOCKS,)
    h_out, w_t, ids_t = pl.pallas_call(
        _fused_kernel,
        grid=grid,
        in_specs=[
            pl.BlockSpec((BLOCK, HIDDEN), lambda i: (i, 0)),
            pl.BlockSpec((NUM_TOKENS, NUM_EXPERTS), lambda i: (0, 0)),
        ],
        out_specs=[
            pl.BlockSpec((BLOCK, HIDDEN), lambda i: (i, 0)),
            pl.BlockSpec((TOP_K, NUM_TOKENS), lambda i: (0, 0)),
            pl.BlockSpec((TOP_K, NUM_TOKENS), lambda i: (0, 0)),
        ],
        out_shape=[
            jax.ShapeDtypeStruct((NUM_TOKENS, HIDDEN), jnp.float32),
            jax.ShapeDtypeStruct((TOP_K, NUM_TOKENS), jnp.float32),
            jax.ShapeDtypeStruct((TOP_K, NUM_TOKENS), jnp.int32),
        ],
    )(hidden_states, router_logits)
    return h_out, w_t.T, ids_t.T


# speedup vs baseline: 1.0125x; 1.0125x over previous
import jax
import jax.numpy as jnp
from jax.experimental import pallas as pl
from jax.experimental.pallas import tpu as pltpu

NUM_TOKENS = 32768
HIDDEN = 2048
NUM_EXPERTS = 64
TOP_K = 8
BLOCK = 1024
N_BLOCKS = NUM_TOKENS // BLOCK


def _fused_kernel(h_ref, logits_ref, h_out_ref, w_ref, id_ref, copy_sem):
    fwd = pltpu.make_async_copy(h_ref, h_out_ref, copy_sem)
    fwd.start()

    i = pl.program_id(0)
    x = logits_ref[pl.ds(i * BLOCK, BLOCK), :]
    xt = x.T
    b = xt.shape[1]
    mx = jnp.max(xt, axis=0, keepdims=True)
    e = jnp.exp(xt - mx)
    probs = e / jnp.sum(e, axis=0, keepdims=True)

    row8 = jax.lax.broadcasted_iota(jnp.int32, (TOP_K, b), 0)
    row64 = jax.lax.broadcasted_iota(jnp.int32, (NUM_EXPERTS, b), 0)
    vals = jnp.zeros((TOP_K, b), dtype=jnp.float32)
    ids = jnp.zeros((TOP_K, b), dtype=jnp.int32)
    cur = probs
    for j in range(TOP_K):
        m = jnp.max(cur, axis=0, keepdims=True)
        a = jnp.argmax(cur, axis=0).astype(jnp.int32)
        a2 = a[None, :]
        vals = jnp.where(row8 == j, m, vals)
        ids = jnp.where(row8 == j, a2, ids)
        cur = jnp.where(row64 == a2, -1.0, cur)
    w = vals / jnp.sum(vals, axis=0, keepdims=True)
    w_ref[:, pl.ds(i * BLOCK, BLOCK)] = w
    id_ref[:, pl.ds(i * BLOCK, BLOCK)] = ids
    fwd.wait()


def kernel(hidden_states, router_logits):
    grid = (N_BLOCKS,)
    h_out, w_t, ids_t = pl.pallas_call(
        _fused_kernel,
        grid=grid,
        in_specs=[
            pl.BlockSpec((BLOCK, HIDDEN), lambda i: (i, 0)),
            pl.BlockSpec((NUM_TOKENS, NUM_EXPERTS), lambda i: (0, 0)),
        ],
        out_specs=[
            pl.BlockSpec((BLOCK, HIDDEN), lambda i: (i, 0)),
            pl.BlockSpec((TOP_K, NUM_TOKENS), lambda i: (0, 0)),
            pl.BlockSpec((TOP_K, NUM_TOKENS), lambda i: (0, 0)),
        ],
        out_shape=[
            jax.ShapeDtypeStruct((NUM_TOKENS, HIDDEN), jnp.float32),
            jax.ShapeDtypeStruct((TOP_K, NUM_TOKENS), jnp.float32),
            jax.ShapeDtypeStruct((TOP_K, NUM_TOKENS), jnp.int32),
        ],
        scratch_shapes=[pltpu.SemaphoreType.DMA],
    )(hidden_states, router_logits)
    return h_out, w_t.T, ids_t.T


# speedup vs baseline: 1.0163x; 1.0037x over previous
import jax
import jax.numpy as jnp
from jax.experimental import pallas as pl

NUM_TOKENS = 32768
HIDDEN = 2048
NUM_EXPERTS = 64
TOP_K = 8
BLOCK = 1024
N_BLOCKS = NUM_TOKENS // BLOCK


def _fused_kernel(h_ref, logits_ref, h_out_ref, w_ref, id_ref):
    h_out_ref[...] = h_ref[...]

    i = pl.program_id(0)
    x = logits_ref[pl.ds(i * BLOCK, BLOCK), :]
    xt = x.T
    b = xt.shape[1]
    mx = jnp.max(xt, axis=0, keepdims=True)
    e = jnp.exp(xt - mx)
    probs = e / jnp.sum(e, axis=0, keepdims=True)

    row8 = jax.lax.broadcasted_iota(jnp.int32, (TOP_K, b), 0)
    row64 = jax.lax.broadcasted_iota(jnp.int32, (NUM_EXPERTS, b), 0)
    vals = jnp.zeros((TOP_K, b), dtype=jnp.float32)
    ids = jnp.zeros((TOP_K, b), dtype=jnp.int32)
    cur = probs
    for j in range(TOP_K):
        m = jnp.max(cur, axis=0, keepdims=True)
        a = jnp.argmax(cur, axis=0).astype(jnp.int32)
        a2 = a[None, :]
        vals = jnp.where(row8 == j, m, vals)
        ids = jnp.where(row8 == j, a2, ids)
        cur = jnp.where(row64 == a2, -1.0, cur)
    w = vals / jnp.sum(vals, axis=0, keepdims=True)
    w_ref[:, pl.ds(i * BLOCK, BLOCK)] = w
    id_ref[:, pl.ds(i * BLOCK, BLOCK)] = ids


def kernel(hidden_states, router_logits):
    grid = (N_BLOCKS,)
    h_out, w_t, ids_t = pl.pallas_call(
        _fused_kernel,
        grid=grid,
        in_specs=[
            pl.BlockSpec((BLOCK, HIDDEN), lambda i: (i, 0)),
            pl.BlockSpec((NUM_TOKENS, NUM_EXPERTS), lambda i: (0, 0)),
        ],
        out_specs=[
            pl.BlockSpec((BLOCK, HIDDEN), lambda i: (i, 0)),
            pl.BlockSpec((TOP_K, NUM_TOKENS), lambda i: (0, 0)),
            pl.BlockSpec((TOP_K, NUM_TOKENS), lambda i: (0, 0)),
        ],
        out_shape=[
            jax.ShapeDtypeStruct((NUM_TOKENS, HIDDEN), jnp.float32),
            jax.ShapeDtypeStruct((TOP_K, NUM_TOKENS), jnp.float32),
            jax.ShapeDtypeStruct((TOP_K, NUM_TOKENS), jnp.int32),
        ],
    )(hidden_states, router_logits)
    return h_out, w_t.T, ids_t.T
